# baseline (speedup 1.0000x reference)
import functools

import jax
import jax.numpy as jnp
from jax import lax
from jax.experimental import pallas as pl
from jax.experimental.pallas import tpu as pltpu
from jax.experimental.pallas import tpu_sc as plsc

G = 50000
H = 10000
NC = 2
NSUB = 16
NW = NC * NSUB
CHUNK = 128
IDXB = 8
ZROWS = 64

H_PAD = 10240
G_PAD = 50176

FRAC0 = 0.5

BLK_G = 1000
BLK_H = 1000


def _cdiv(a, b):
    return -(-a // b)


def _prep_edges(src, dst, dummy, dummy_spread, n_src):
    e = src.shape[0]
    totc = _cdiv(e, NSUB * IDXB * CHUNK) * NSUB * IDXB
    epad = totc * CHUNK
    npad = epad - e
    ar = jnp.arange(npad, dtype=jnp.int32)
    src = jnp.concatenate([src.astype(jnp.int32),
                           ar % jnp.int32(n_src)])
    dst = jnp.concatenate([dst.astype(jnp.int32),
                           jnp.int32(dummy) + ar % jnp.int32(dummy_spread)])
    return src.reshape(totc, CHUNK), dst.reshape(totc, CHUNK)


@functools.lru_cache(maxsize=None)
def _seg_sum_kernel(n_tables, totc, n_dst_pad, d, frac0, nbuf):
    mesh = plsc.VectorSubcoreMesh(core_axis_name="c", subcore_axis_name="s",
                                  num_cores=NC, num_subcores=NSUB)
    rps = n_dst_pad // NSUB
    s_chunks = totc // NSUB
    n0 = int(round(s_chunks * frac0 / IDXB)) * IDXB
    n1 = s_chunks - n0
    assert n0 > 0 and n1 > 0 and n1 % IDXB == 0

    def body(*refs):
        tables = refs[:n_tables]
        (sidx, didx, out, idx_s, idx_d, rows, zbuf, acc,
         gsem, ssem, isem, zsem) = refs[n_tables:]
        cid = lax.axis_index("c")
        sid = lax.axis_index("s")
        base = jnp.where(cid == 0, sid * n0, NSUB * n0 + sid * n1)
        nbw = jnp.where(cid == 0, n0 // IDXB, n1 // IDXB)
        zero = jnp.zeros((16,), jnp.float32)

        @pl.loop(0, ZROWS)
        def _(r):
            for k in range(d // 16):
                zbuf[r, pl.ds(k * 16, 16)] = zero

        row0 = sid * rps

        def gather_start(t, cb, j, b):
            pltpu.async_copy(tables[t].at[idx_s.at[cb, j]], rows.at[b],
                             gsem.at[b])

        def gather_wait(t, cb, j, b):
            pltpu.make_async_copy(tables[t].at[idx_s.at[cb, j]], rows.at[b],
                                  gsem.at[b]).wait()

        def scatter_start(cb, j, b):
            pltpu.async_copy(rows.at[b], acc.at[idx_d.at[cb, j]],
                             ssem.at[b], add=True)

        def scatter_wait(cb, j, b):
            pltpu.make_async_copy(rows.at[b], acc.at[idx_d.at[cb, j]],
                                  ssem.at[b]).wait()

        for t in range(n_tables):
            @pl.loop(0, rps // ZROWS)
            def _(j):
                pltpu.async_copy(zbuf, acc.at[pl.ds(row0 + j * ZROWS, ZROWS)],
                                 zsem)

            @pl.loop(0, rps // ZROWS)
            def _(j):
                pltpu.make_async_copy(zbuf, acc.at[pl.ds(row0, ZROWS)],
                                      zsem).wait()

            plsc.subcore_barrier()
            lk = nbuf // 2
            slack = nbuf - lk
            pltpu.sync_copy(sidx.at[pl.ds(base, IDXB)], idx_s.at[0])
            pltpu.sync_copy(didx.at[pl.ds(base, IDXB)], idx_d.at[0])
            for j in range(lk):
                gather_start(t, 0, j, j)

            @pl.loop(0, nbw)
            def _(jb):
                cur = jb % 2
                nxt = (jb + 1) % 2

                @pl.when(jb + 1 < nbw)
                def _():
                    pltpu.async_copy(
                        sidx.at[pl.ds(base + (jb + 1) * IDXB, IDXB)],
                        idx_s.at[nxt], isem)
                    pltpu.async_copy(
                        didx.at[pl.ds(base + (jb + 1) * IDXB, IDXB)],
                        idx_d.at[nxt], isem)

                for j in range(IDXB):
                    b = j % nbuf
                    gather_wait(t, cur, j, b)
                    ob = (j + lk) % nbuf
                    if j < slack:
                        @pl.when(jb > 0)
                        def _():
                            scatter_wait(cur, j, ob)
                    else:
                        scatter_wait(cur, j, ob)
                    scatter_start(cur, j, b)
                    jn = j + lk
                    if jn < IDXB:
                        gather_start(t, cur, jn, jn % nbuf)
                    else:
                        if jn == IDXB:
                            @pl.when(jb + 1 < nbw)
                            def _():
                                pltpu.make_async_copy(
                                    sidx.at[pl.ds(base, IDXB)],
                                    idx_s.at[nxt], isem).wait()
                                pltpu.make_async_copy(
                                    didx.at[pl.ds(base, IDXB)],
                                    idx_d.at[nxt], isem).wait()

                        @pl.when(jb + 1 < nbw)
                        def _():
                            gather_start(t, nxt, jn - IDXB, jn % nbuf)

            for k in range(1, slack + 1):
                scatter_wait(0, IDXB - k, (IDXB - k) % nbuf)
            plsc.subcore_barrier()
            pltpu.sync_copy(acc.at[pl.ds(row0, rps)],
                            out.at[cid, t, pl.ds(row0, rps)])
            plsc.subcore_barrier()

    return pl.kernel(
        body,
        out_type=jax.ShapeDtypeStruct((NC, n_tables, n_dst_pad, d),
                                      jnp.float32),
        mesh=mesh,
        scratch_types=[
            pltpu.VMEM((2, IDXB, CHUNK), jnp.int32),
            pltpu.VMEM((2, IDXB, CHUNK), jnp.int32),
            pltpu.VMEM((nbuf, CHUNK, d), jnp.float32),
            pltpu.VMEM((ZROWS, d), jnp.float32),
            pltpu.VMEM_SHARED((n_dst_pad, d), jnp.float32),
            pltpu.SemaphoreType.DMA((nbuf,)),
            pltpu.SemaphoreType.DMA((nbuf,)),
            pltpu.SemaphoreType.DMA,
            pltpu.SemaphoreType.DMA,
        ],
        compiler_params=pltpu.CompilerParams(use_tc_tiling_on_sc=(d == 128)),
    )


def _dot(a, b):
    return jnp.dot(a, b, preferred_element_type=jnp.float32,
                   precision=lax.Precision.DEFAULT)


def _enc_dense(x, attr_data, w_cat):
    def body(x_ref, ad_ref, wc_ref, p_ref, xf_ref):
        xv = jnp.concatenate([x_ref[0, 0, 0], x_ref[0, 1, 0], ad_ref[...]],
                             axis=1)
        ps = []
        for i in range(4):
            y = _dot(xv, wc_ref[i])
            ps.append(y[:, :32])
            xv = xv + jnp.maximum(y[:, 32:], 0.0)
        p_ref[...] = jnp.concatenate(ps, axis=1)
        xf_ref[...] = xv

    return pl.pallas_call(
        body,
        grid=(G // BLK_G,),
        in_specs=[
            pl.BlockSpec((1, 2, 1, BLK_G, 32), lambda i: (0, 0, 0, i, 0)),
            pl.BlockSpec((BLK_G, 4), lambda i: (i, 0)),
            pl.BlockSpec((4, 68, 100), lambda i: (0, 0, 0)),
        ],
        out_specs=[
            pl.BlockSpec((BLK_G, 128), lambda i: (i, 0)),
            pl.BlockSpec((BLK_G, 68), lambda i: (i, 0)),
        ],
        out_shape=[
            jax.ShapeDtypeStruct((G, 128), jnp.float32),
            jax.ShapeDtypeStruct((G, 68), jnp.float32),
        ],
    )(x, attr_data, w_cat)


def _post_enc(agg, attr_hidden, wdst_cat, wp0):
    def body(agg_ref, ah_ref, wd_ref, wp_ref, xl_ref, p_ref):
        a = agg_ref[0, 0] + agg_ref[1, 0]
        xl = jnp.maximum(a + _dot(ah_ref[...], wd_ref[...]), 0.0)
        xl_ref[...] = xl
        p_ref[...] = _dot(xl, wp_ref[...])

    return pl.pallas_call(
        body,
        grid=(H // BLK_H,),
        in_specs=[
            pl.BlockSpec((NC, 1, BLK_H, 128), lambda i: (0, 0, i, 0)),
            pl.BlockSpec((BLK_H, 4), lambda i: (i, 0)),
            pl.BlockSpec((4, 128), lambda i: (0, 0)),
            pl.BlockSpec((128, 128), lambda i: (0, 0)),
        ],
        out_specs=[
            pl.BlockSpec((BLK_H, 128), lambda i: (i, 0)),
            pl.BlockSpec((BLK_H, 128), lambda i: (i, 0)),
        ],
        out_shape=[
            jax.ShapeDtypeStruct((H, 128), jnp.float32),
            jax.ShapeDtypeStruct((H, 128), jnp.float32),
        ],
    )(agg, attr_hidden, wdst_cat, wp0)


def _proc_step(h, agg, addin, wself, wnext):
    def body(h_ref, agg_ref, ad_ref, ws_ref, wn_ref, hn_ref, p_ref):
        hv = h_ref[...]
        a = agg_ref[0, 0] + agg_ref[1, 0]
        hn = hv + jnp.maximum(a + _dot(hv, ws_ref[...]), 0.0)
        hn_ref[...] = hn
        p_ref[...] = _dot(hn + ad_ref[...], wn_ref[...])

    return pl.pallas_call(
        body,
        grid=(H // BLK_H,),
        in_specs=[
            pl.BlockSpec((BLK_H, 128), lambda i: (i, 0)),
            pl.BlockSpec((NC, 1, BLK_H, 128), lambda i: (0, 0, i, 0)),
            pl.BlockSpec((BLK_H, 128), lambda i: (i, 0)),
            pl.BlockSpec((128, 128), lambda i: (0, 0)),
            pl.BlockSpec((128, 128), lambda i: (0, 0)),
        ],
        out_specs=[
            pl.BlockSpec((BLK_H, 128), lambda i: (i, 0)),
            pl.BlockSpec((BLK_H, 128), lambda i: (i, 0)),
        ],
        out_shape=[
            jax.ShapeDtypeStruct((H, 128), jnp.float32),
            jax.ShapeDtypeStruct((H, 128), jnp.float32),
        ],
    )(h, agg, addin, wself, wnext)


def _dstp(xf, wd_dst_cat):
    def body(xf_ref, w_ref, o_ref):
        o_ref[...] = _dot(xf_ref[...], w_ref[...])

    return pl.pallas_call(
        body,
        grid=(G // BLK_G,),
        in_specs=[
            pl.BlockSpec((BLK_G, 68), lambda i: (i, 0)),
            pl.BlockSpec((68, 128), lambda i: (0, 0)),
        ],
        out_specs=pl.BlockSpec((BLK_G, 128), lambda i: (i, 0)),
        out_shape=jax.ShapeDtypeStruct((G, 128), jnp.float32),
    )(xf, wd_dst_cat)


def _post_dec(aggd, dstp, wd_out1, carry, col):
    def body(agg_ref, dp_ref, wdo_ref, c_ref, y_ref):
        acc = c_ref[...]
        for hh in range(2):
            a = agg_ref[0, hh] + agg_ref[1, hh]
            c0 = 64 * col + 32 * hh
            r = jnp.maximum(a + dp_ref[:, c0:c0 + 32], 0.0)
            acc = acc + _dot(r, wdo_ref[hh])
        y_ref[...] = acc

    return pl.pallas_call(
        body,
        grid=(G // BLK_G,),
        in_specs=[
            pl.BlockSpec((NC, 2, BLK_G, 32), lambda i: (0, 0, i, 0)),
            pl.BlockSpec((BLK_G, 128), lambda i: (i, 0)),
            pl.BlockSpec((2, 32, 32), lambda i: (0, 0, 0)),
            pl.BlockSpec((BLK_G, 32), lambda i: (i, 0)),
        ],
        out_specs=pl.BlockSpec((BLK_G, 32), lambda i: (i, 0)),
        out_shape=jax.ShapeDtypeStruct((G, 32), jnp.float32),
    )(aggd, dstp, wd_out1, carry)


def kernel(x, attr_data, attr_hidden, W_enc_src, W_enc_dst, W_upd,
           Wp_src, Wp_self, Wd_src, Wd_dst, Wd_out,
           e_src_enc, e_dst_enc, e_src_proc, e_dst_proc,
           e_src_dec, e_dst_dec):
    b = x.shape[0]
    ens = x.shape[2]
    v = x.shape[4]
    xlast = x[:, -1].reshape(G, v)

    se, de = _prep_edges(e_src_enc, e_dst_enc, H, H_PAD - H, G)
    sp, dp = _prep_edges(e_src_proc, e_dst_proc, H, H_PAD - H, H)
    sd, dd = _prep_edges(e_src_dec, e_dst_dec, G, G_PAD - G, H)

    w_cat = jnp.concatenate([W_enc_src, W_upd], axis=2)
    p_enc, xf = _enc_dense(x, attr_data, w_cat)
    agg_e = _seg_sum_kernel(1, se.shape[0], H_PAD, 128, FRAC0, 2)(p_enc, se, de)
    wdst_cat = jnp.transpose(W_enc_dst, (1, 0, 2)).reshape(4, 128)
    xl, p0 = _post_enc(agg_e, attr_hidden, wdst_cat, Wp_src[0])

    seg_p = _seg_sum_kernel(1, sp.shape[0], H_PAD, 128, FRAC0, 2)
    agg_p0 = seg_p(p0, sp, dp)
    zeros_h = jnp.zeros((H, 128), jnp.float32)
    wd_blk = jax.scipy.linalg.block_diag(Wd_src[0], Wd_src[1],
                                         Wd_src[2], Wd_src[3])
    h1, p1 = _proc_step(xl, agg_p0, zeros_h, Wp_self[0], Wp_src[1])
    agg_p1 = seg_p(p1, sp, dp)
    _, tdec = _proc_step(h1, agg_p1, xl, Wp_self[1], wd_blk)

    tabs = [lax.slice_in_dim(tdec, 32 * i, 32 * (i + 1), axis=1)
            for i in range(4)]
    wd_dst_cat = jnp.transpose(Wd_dst, (1, 0, 2)).reshape(68, 128)
    dstp = _dstp(xf, wd_dst_cat)
    seg_d = _seg_sum_kernel(2, sd.shape[0], G_PAD, 32, FRAC0, 4)
    agg_da = seg_d(tabs[0], tabs[1], sd, dd)
    agg_db = seg_d(tabs[2], tabs[3], sd, dd)
    y = _post_dec(agg_da, dstp, Wd_out[:2], xlast, 0)
    y = _post_dec(agg_db, dstp, Wd_out[2:], y, 1)
    return y.reshape(b, ens, G, v)


# speedup vs baseline: 1.0179x; 1.0179x over previous
import functools

import jax
import jax.numpy as jnp
from jax import lax
from jax.experimental import pallas as pl
from jax.experimental.pallas import tpu as pltpu
from jax.experimental.pallas import tpu_sc as plsc

G = 50000
H = 10000
NC = 2
NSUB = 16
NW = NC * NSUB
CHUNK = 128
IDXB = 8
ZROWS = 64

H_PAD = 10240
G_PAD = 50176

FRAC0 = 0.5

BLK_G = 1000
BLK_H = 1000


def _cdiv(a, b):
    return -(-a // b)


def _prep_edges(src, dst, dummy, dummy_spread, n_src):
    e = src.shape[0]
    totc = _cdiv(e, NSUB * IDXB * CHUNK) * NSUB * IDXB
    epad = totc * CHUNK
    npad = epad - e
    ar = jnp.arange(npad, dtype=jnp.int32)
    src = jnp.concatenate([src.astype(jnp.int32),
                           ar % jnp.int32(n_src)])
    dst = jnp.concatenate([dst.astype(jnp.int32),
                           jnp.int32(dummy) + ar % jnp.int32(dummy_spread)])
    return src.reshape(totc, CHUNK), dst.reshape(totc, CHUNK)


@functools.lru_cache(maxsize=None)
def _seg_sum_kernel(n_tables, totc, n_dst_pad, d, frac0, nbuf):
    mesh = plsc.VectorSubcoreMesh(core_axis_name="c", subcore_axis_name="s",
                                  num_cores=NC, num_subcores=NSUB)
    rps = n_dst_pad // NSUB
    s_chunks = totc // NSUB
    n0 = int(round(s_chunks * frac0 / IDXB)) * IDXB
    n1 = s_chunks - n0
    assert n0 > 0 and n1 > 0 and n1 % IDXB == 0

    def body(*refs):
        tables = refs[:n_tables]
        (sidx, didx, out, idx_s, idx_d, rows, zbuf, acc,
         gsem, ssem, isem, zsem) = refs[n_tables:]
        cid = lax.axis_index("c")
        sid = lax.axis_index("s")
        base = jnp.where(cid == 0, sid * n0, NSUB * n0 + sid * n1)
        nbw = jnp.where(cid == 0, n0 // IDXB, n1 // IDXB)
        zero = jnp.zeros((16,), jnp.float32)

        @pl.loop(0, ZROWS)
        def _(r):
            for k in range(d // 16):
                zbuf[r, pl.ds(k * 16, 16)] = zero

        row0 = sid * rps

        def gather_start(t, cb, j, b):
            pltpu.async_copy(tables[t].at[idx_s.at[cb, j]], rows.at[b],
                             gsem.at[b])

        def gather_wait(t, cb, j, b):
            pltpu.make_async_copy(tables[t].at[idx_s.at[cb, j]], rows.at[b],
                                  gsem.at[b]).wait()

        def scatter_start(cb, j, b):
            pltpu.async_copy(rows.at[b], acc.at[idx_d.at[cb, j]],
                             ssem.at[b], add=True)

        def scatter_wait(cb, j, b):
            pltpu.make_async_copy(rows.at[b], acc.at[idx_d.at[cb, j]],
                                  ssem.at[b]).wait()

        for t in range(n_tables):
            @pl.loop(0, rps // ZROWS)
            def _(j):
                pltpu.async_copy(zbuf, acc.at[pl.ds(row0 + j * ZROWS, ZROWS)],
                                 zsem)

            @pl.loop(0, rps // ZROWS)
            def _(j):
                pltpu.make_async_copy(zbuf, acc.at[pl.ds(row0, ZROWS)],
                                      zsem).wait()

            plsc.subcore_barrier()
            lk = nbuf // 2
            slack = nbuf - lk
            pltpu.sync_copy(sidx.at[pl.ds(base, IDXB)], idx_s.at[0])
            pltpu.sync_copy(didx.at[pl.ds(base, IDXB)], idx_d.at[0])
            for j in range(lk):
                gather_start(t, 0, j, j)

            @pl.loop(0, nbw)
            def _(jb):
                cur = jb % 2
                nxt = (jb + 1) % 2

                @pl.when(jb + 1 < nbw)
                def _():
                    pltpu.async_copy(
                        sidx.at[pl.ds(base + (jb + 1) * IDXB, IDXB)],
                        idx_s.at[nxt], isem)
                    pltpu.async_copy(
                        didx.at[pl.ds(base + (jb + 1) * IDXB, IDXB)],
                        idx_d.at[nxt], isem)

                for j in range(IDXB):
                    b = j % nbuf
                    gather_wait(t, cur, j, b)
                    ob = (j + lk) % nbuf
                    if j < slack:
                        @pl.when(jb > 0)
                        def _():
                            scatter_wait(cur, j, ob)
                    else:
                        scatter_wait(cur, j, ob)
                    scatter_start(cur, j, b)
                    jn = j + lk
                    if jn < IDXB:
                        gather_start(t, cur, jn, jn % nbuf)
                    else:
                        if jn == IDXB:
                            @pl.when(jb + 1 < nbw)
                            def _():
                                pltpu.make_async_copy(
                                    sidx.at[pl.ds(base, IDXB)],
                                    idx_s.at[nxt], isem).wait()
                                pltpu.make_async_copy(
                                    didx.at[pl.ds(base, IDXB)],
                                    idx_d.at[nxt], isem).wait()

                        @pl.when(jb + 1 < nbw)
                        def _():
                            gather_start(t, nxt, jn - IDXB, jn % nbuf)

            for k in range(1, slack + 1):
                scatter_wait(0, IDXB - k, (IDXB - k) % nbuf)
            plsc.subcore_barrier()
            pltpu.sync_copy(acc.at[pl.ds(row0, rps)],
                            out.at[cid, t, pl.ds(row0, rps)])
            plsc.subcore_barrier()

    return pl.kernel(
        body,
        out_type=jax.ShapeDtypeStruct((NC, n_tables, n_dst_pad, d),
                                      jnp.float32),
        mesh=mesh,
        scratch_types=[
            pltpu.VMEM((2, IDXB, CHUNK), jnp.int32),
            pltpu.VMEM((2, IDXB, CHUNK), jnp.int32),
            pltpu.VMEM((nbuf, CHUNK, d), jnp.float32),
            pltpu.VMEM((ZROWS, d), jnp.float32),
            pltpu.VMEM_SHARED((n_dst_pad, d), jnp.float32),
            pltpu.SemaphoreType.DMA((nbuf,)),
            pltpu.SemaphoreType.DMA((nbuf,)),
            pltpu.SemaphoreType.DMA,
            pltpu.SemaphoreType.DMA,
        ],
        compiler_params=pltpu.CompilerParams(use_tc_tiling_on_sc=(d == 128)),
    )


def _dot(a, b):
    return jnp.dot(a, b, preferred_element_type=jnp.float32,
                   precision=lax.Precision.DEFAULT)


def _enc_dense(x0, w_cat):
    def body(x_ref, wc_ref, p_ref, xf_ref):
        xv = x_ref[...]
        ps = []
        for i in range(4):
            y = _dot(xv, wc_ref[i])
            ps.append(y[:, :32])
            xv = xv + jnp.maximum(y[:, 32:], 0.0)
        p_ref[...] = jnp.concatenate(ps, axis=1)
        xf_ref[...] = xv

    return pl.pallas_call(
        body,
        grid=(G // BLK_G,),
        in_specs=[
            pl.BlockSpec((BLK_G, 68), lambda i: (i, 0)),
            pl.BlockSpec((4, 68, 100), lambda i: (0, 0, 0)),
        ],
        out_specs=[
            pl.BlockSpec((BLK_G, 128), lambda i: (i, 0)),
            pl.BlockSpec((BLK_G, 68), lambda i: (i, 0)),
        ],
        out_shape=[
            jax.ShapeDtypeStruct((G, 128), jnp.float32),
            jax.ShapeDtypeStruct((G, 68), jnp.float32),
        ],
    )(x0, w_cat)


def _post_enc(agg, attr_hidden, wdst_cat, wp0):
    def body(agg_ref, ah_ref, wd_ref, wp_ref, xl_ref, p_ref):
        a = agg_ref[0, 0] + agg_ref[1, 0]
        xl = jnp.maximum(a + _dot(ah_ref[...], wd_ref[...]), 0.0)
        xl_ref[...] = xl
        p_ref[...] = _dot(xl, wp_ref[...])

    return pl.pallas_call(
        body,
        grid=(H // BLK_H,),
        in_specs=[
            pl.BlockSpec((NC, 1, BLK_H, 128), lambda i: (0, 0, i, 0)),
            pl.BlockSpec((BLK_H, 4), lambda i: (i, 0)),
            pl.BlockSpec((4, 128), lambda i: (0, 0)),
            pl.BlockSpec((128, 128), lambda i: (0, 0)),
        ],
        out_specs=[
            pl.BlockSpec((BLK_H, 128), lambda i: (i, 0)),
            pl.BlockSpec((BLK_H, 128), lambda i: (i, 0)),
        ],
        out_shape=[
            jax.ShapeDtypeStruct((H, 128), jnp.float32),
            jax.ShapeDtypeStruct((H, 128), jnp.float32),
        ],
    )(agg, attr_hidden, wdst_cat, wp0)


def _proc_step(h, agg, addin, wself, wnext):
    def body(h_ref, agg_ref, ad_ref, ws_ref, wn_ref, hn_ref, p_ref):
        hv = h_ref[...]
        a = agg_ref[0, 0] + agg_ref[1, 0]
        hn = hv + jnp.maximum(a + _dot(hv, ws_ref[...]), 0.0)
        hn_ref[...] = hn
        p_ref[...] = _dot(hn + ad_ref[...], wn_ref[...])

    return pl.pallas_call(
        body,
        grid=(H // BLK_H,),
        in_specs=[
            pl.BlockSpec((BLK_H, 128), lambda i: (i, 0)),
            pl.BlockSpec((NC, 1, BLK_H, 128), lambda i: (0, 0, i, 0)),
            pl.BlockSpec((BLK_H, 128), lambda i: (i, 0)),
            pl.BlockSpec((128, 128), lambda i: (0, 0)),
            pl.BlockSpec((128, 128), lambda i: (0, 0)),
        ],
        out_specs=[
            pl.BlockSpec((BLK_H, 128), lambda i: (i, 0)),
            pl.BlockSpec((BLK_H, 128), lambda i: (i, 0)),
        ],
        out_shape=[
            jax.ShapeDtypeStruct((H, 128), jnp.float32),
            jax.ShapeDtypeStruct((H, 128), jnp.float32),
        ],
    )(h, agg, addin, wself, wnext)


def _dstp(xf, wd_dst_cat):
    def body(xf_ref, w_ref, o_ref):
        o_ref[...] = _dot(xf_ref[...], w_ref[...])

    return pl.pallas_call(
        body,
        grid=(G // BLK_G,),
        in_specs=[
            pl.BlockSpec((BLK_G, 68), lambda i: (i, 0)),
            pl.BlockSpec((68, 128), lambda i: (0, 0)),
        ],
        out_specs=pl.BlockSpec((BLK_G, 128), lambda i: (i, 0)),
        out_shape=jax.ShapeDtypeStruct((G, 128), jnp.float32),
    )(xf, wd_dst_cat)


def _post_dec(aggd, dstp, wd_out1, carry, col):
    def body(agg_ref, dp_ref, wdo_ref, c_ref, y_ref):
        acc = c_ref[...]
        for hh in range(2):
            a = agg_ref[0, hh] + agg_ref[1, hh]
            c0 = 64 * col + 32 * hh
            r = jnp.maximum(a + dp_ref[:, c0:c0 + 32], 0.0)
            acc = acc + _dot(r, wdo_ref[hh])
        y_ref[...] = acc

    return pl.pallas_call(
        body,
        grid=(G // BLK_G,),
        in_specs=[
            pl.BlockSpec((NC, 2, BLK_G, 32), lambda i: (0, 0, i, 0)),
            pl.BlockSpec((BLK_G, 128), lambda i: (i, 0)),
            pl.BlockSpec((2, 32, 32), lambda i: (0, 0, 0)),
            pl.BlockSpec((BLK_G, 32), lambda i: (i, 0)),
        ],
        out_specs=pl.BlockSpec((BLK_G, 32), lambda i: (i, 0)),
        out_shape=jax.ShapeDtypeStruct((G, 32), jnp.float32),
    )(aggd, dstp, wd_out1, carry)


def kernel(x, attr_data, attr_hidden, W_enc_src, W_enc_dst, W_upd,
           Wp_src, Wp_self, Wd_src, Wd_dst, Wd_out,
           e_src_enc, e_dst_enc, e_src_proc, e_dst_proc,
           e_src_dec, e_dst_dec):
    b = x.shape[0]
    ens = x.shape[2]
    v = x.shape[4]
    x_r = jnp.transpose(x, (0, 2, 3, 1, 4)).reshape(G, x.shape[1] * v)
    x0 = jnp.concatenate([x_r, attr_data], axis=-1)
    xlast = x[:, -1].reshape(G, v)

    se, de = _prep_edges(e_src_enc, e_dst_enc, H, H_PAD - H, G)
    sp, dp = _prep_edges(e_src_proc, e_dst_proc, H, H_PAD - H, H)
    sd, dd = _prep_edges(e_src_dec, e_dst_dec, G, G_PAD - G, H)

    w_cat = jnp.concatenate([W_enc_src, W_upd], axis=2)
    p_enc, xf = _enc_dense(x0, w_cat)
    agg_e = _seg_sum_kernel(1, se.shape[0], H_PAD, 128, FRAC0, 2)(p_enc, se, de)
    wdst_cat = jnp.transpose(W_enc_dst, (1, 0, 2)).reshape(4, 128)
    xl, p0 = _post_enc(agg_e, attr_hidden, wdst_cat, Wp_src[0])

    seg_p = _seg_sum_kernel(1, sp.shape[0], H_PAD, 128, FRAC0, 2)
    agg_p0 = seg_p(p0, sp, dp)
    zeros_h = jnp.zeros((H, 128), jnp.float32)
    wd_blk = jax.scipy.linalg.block_diag(Wd_src[0], Wd_src[1],
                                         Wd_src[2], Wd_src[3])
    h1, p1 = _proc_step(xl, agg_p0, zeros_h, Wp_self[0], Wp_src[1])
    agg_p1 = seg_p(p1, sp, dp)
    _, tdec = _proc_step(h1, agg_p1, xl, Wp_self[1], wd_blk)

    tabs = [lax.slice_in_dim(tdec, 32 * i, 32 * (i + 1), axis=1)
            for i in range(4)]
    wd_dst_cat = jnp.transpose(Wd_dst, (1, 0, 2)).reshape(68, 128)
    dstp = _dstp(xf, wd_dst_cat)
    seg_d = _seg_sum_kernel(2, sd.shape[0], G_PAD, 32, FRAC0, 4)
    agg_da = seg_d(tabs[0], tabs[1], sd, dd)
    agg_db = seg_d(tabs[2], tabs[3], sd, dd)
    y = _post_dec(agg_da, dstp, Wd_out[:2], xlast, 0)
    y = _post_dec(agg_db, dstp, Wd_out[2:], y, 1)
    return y.reshape(b, ens, G, v)
